# baseline (speedup 1.0000x reference)
import functools

import jax
import jax.numpy as jnp
from jax import lax
from jax.experimental import pallas as pl
from jax.experimental.pallas import tpu as pltpu
from jax.experimental.pallas import tpu_sc as plsc

NW = 32
ROWS_PER_STEP = 2


def kernel(cell_index, cell_means):
    B = cell_index.shape[0]
    V, D = cell_means.shape
    b_per_w = B // NW
    n_steps = b_per_w // ROWS_PER_STEP

    idx = cell_index.astype(jnp.int32).reshape(NW, n_steps, ROWS_PER_STEP)

    mesh = plsc.VectorSubcoreMesh(core_axis_name="c", subcore_axis_name="s")

    @functools.partial(
        pl.kernel,
        mesh=mesh,
        out_type=jax.ShapeDtypeStruct((B, D), jnp.float32),
        compiler_params=pltpu.CompilerParams(use_tc_tiling_on_sc=False),
        scratch_types=[
            pltpu.VMEM((n_steps, ROWS_PER_STEP), jnp.int32),
            pltpu.VMEM((ROWS_PER_STEP, D), jnp.float32),
            pltpu.SemaphoreType.DMA,
        ],
    )
    def gather_kernel(idx_hbm, table_hbm, out_hbm, idx_v, rows_v, sem):
        wid = lax.axis_index("s") * 2 + lax.axis_index("c")
        base = wid * b_per_w
        pltpu.sync_copy(idx_hbm.at[wid], idx_v)

        def body(i, carry):
            pltpu.async_copy(table_hbm.at[idx_v.at[i]], rows_v, sem).wait()
            pltpu.sync_copy(
                rows_v, out_hbm.at[pl.ds(base + i * ROWS_PER_STEP, ROWS_PER_STEP)]
            )
            return carry

        lax.fori_loop(0, n_steps, body, 0)

    return gather_kernel(idx, cell_means)


# speedup vs baseline: 1.0474x; 1.0474x over previous
import functools

import jax
import jax.numpy as jnp
from jax import lax
from jax.experimental import pallas as pl
from jax.experimental.pallas import tpu as pltpu
from jax.experimental.pallas import tpu_sc as plsc

NW = 32
ROWS_PER_STEP = 2


def kernel(cell_index, cell_means):
    B = cell_index.shape[0]
    V, D = cell_means.shape
    b_per_w = B // NW
    n_steps = b_per_w // ROWS_PER_STEP

    idx = cell_index.astype(jnp.int32).reshape(NW, n_steps, ROWS_PER_STEP)

    mesh = plsc.VectorSubcoreMesh(core_axis_name="c", subcore_axis_name="s")

    @functools.partial(
        pl.kernel,
        mesh=mesh,
        out_type=jax.ShapeDtypeStruct((B, D), jnp.float32),
        compiler_params=pltpu.CompilerParams(use_tc_tiling_on_sc=False),
        scratch_types=[
            pltpu.VMEM((n_steps, ROWS_PER_STEP), jnp.int32),
            pltpu.VMEM((ROWS_PER_STEP, D), jnp.float32),
            pltpu.VMEM((ROWS_PER_STEP, D), jnp.float32),
            pltpu.SemaphoreType.DMA,
            pltpu.SemaphoreType.DMA,
        ],
    )
    def gather_kernel(idx_hbm, table_hbm, out_hbm, idx_v, buf0, buf1, gs0, gs1):
        wid = lax.axis_index("s") * 2 + lax.axis_index("c")
        base = wid * b_per_w
        pltpu.sync_copy(idx_hbm.at[wid], idx_v)

        bufs = (buf0, buf1)
        gsems = (gs0, gs1)

        def g_start(step, b):
            pltpu.async_copy(table_hbm.at[idx_v.at[step]], bufs[b], gsems[b])

        g_start(0, 0)
        g_start(1, 1)

        def body(i2, carry):
            for b in range(2):
                step = i2 * 2 + b
                pltpu.make_async_copy(
                    table_hbm.at[idx_v.at[step]], bufs[b], gsems[b]
                ).wait()
                pltpu.sync_copy(
                    bufs[b],
                    out_hbm.at[pl.ds(base + step * ROWS_PER_STEP, ROWS_PER_STEP)],
                )

                @pl.when(step + 2 < n_steps)
                def _():
                    g_start(step + 2, b)

            return carry

        lax.fori_loop(0, n_steps // 2, body, 0)

    return gather_kernel(idx, cell_means)
